# baseline (speedup 1.0000x reference)
import functools
import math

import jax
import jax.numpy as jnp
from jax import lax
from jax.experimental import pallas as pl
from jax.experimental.pallas import tpu as pltpu


def _tsum_body(x_ref, xs_ref):
    xs_ref[0] = jnp.sum(x_ref[0], axis=-1)


def _main_body(xs_ref, out_ref, *, n_rows, n, c, k, n_iters):
    j = pl.program_id(1)
    xs = xs_ref[0]
    lhs = xs_ref[0, :, pl.ds(j * n_rows, n_rows)]
    a = lax.dot_general(lhs, xs, (((0,), (0,)), ((), ())),
                        preferred_element_type=jnp.float32)
    a = a / math.sqrt(c)
    r = jnp.maximum(a, 0.0)
    e1 = jnp.exp(r - jnp.max(r, axis=-1, keepdims=True))
    s = e1 / jnp.sum(e1, axis=-1, keepdims=True)
    e2 = jnp.exp(s - jnp.max(s, axis=-1, keepdims=True))
    w = e2 / jnp.sum(e2, axis=-1, keepdims=True)

    bits = lax.bitcast_convert_type(w, jnp.int32)

    zero_w = jnp.max(jnp.where(r == 0.0, w, 0.0), axis=-1, keepdims=True)
    t0 = lax.bitcast_convert_type(zero_w, jnp.int32)
    gp = jnp.sum((bits > t0).astype(jnp.int32), axis=-1, keepdims=True)

    def full_search():
        def search(i, carry):
            lo, hi = carry
            mid = (lo + hi) >> 1
            cnt = jnp.sum((bits >= mid).astype(jnp.int32), axis=-1,
                          keepdims=True)
            ge = cnt >= k
            return jnp.where(ge, mid, lo), jnp.where(ge, hi, mid)

        lo0 = jnp.zeros((n_rows, 1), jnp.int32)
        hi0 = jnp.full((n_rows, 1), 0x3F800001, jnp.int32)
        t = lax.fori_loop(0, n_iters, search, (lo0, hi0))[0]
        g = jnp.sum((bits > t).astype(jnp.int32), axis=-1, keepdims=True)
        return t, g

    t, g = lax.cond(jnp.any(gp >= k), full_search, lambda: (t0, gp))

    gt = bits > t
    eq = bits == t
    z = eq.astype(jnp.int32)
    cum = z
    sh = 1
    while sh < n:
        cum = cum + lax.concatenate(
            [jnp.zeros((n_rows, sh), jnp.int32), cum[:, : n - sh]], 1)
        sh *= 2
    pc = cum - z
    keep = gt | (eq & (pc < (k - g)))
    out_ref[0] = jnp.where(keep, w, 0.0)


def kernel(x):
    b, c, n, t = x.shape
    k = int(n * 0.8)
    n_rows = 512 if n % 512 == 0 else n
    c_chunk = 8 if c % 8 == 0 else c
    xs = pl.pallas_call(
        _tsum_body,
        grid=(b, c // c_chunk),
        in_specs=[pl.BlockSpec((1, c_chunk, n, t),
                               lambda bi, ci: (bi, ci, 0, 0))],
        out_specs=pl.BlockSpec((1, c_chunk, n), lambda bi, ci: (bi, ci, 0)),
        out_shape=jax.ShapeDtypeStruct((b, c, n), jnp.float32),
    )(x)
    body = functools.partial(_main_body, n_rows=n_rows, n=n, c=c, k=k,
                             n_iters=30)
    return pl.pallas_call(
        body,
        grid=(b, n // n_rows),
        in_specs=[pl.BlockSpec((1, c, n), lambda bi, ji: (bi, 0, 0))],
        out_specs=pl.BlockSpec((1, n_rows, n), lambda bi, ji: (bi, ji, 0)),
        out_shape=jax.ShapeDtypeStruct((b, n, n), jnp.float32),
    )(xs)


# speedup vs baseline: 2.2446x; 2.2446x over previous
import functools
import math

import jax
import jax.numpy as jnp
from jax import lax
from jax.experimental import pallas as pl
from jax.experimental.pallas import tpu as pltpu


def _body(x_ref, out_ref, xs_ref, *, n_rows, n, c, k, n_iters):
    j = pl.program_id(1)

    @pl.when(j == 0)
    def _():
        xs_ref[...] = jnp.sum(x_ref[0], axis=0)

    xs = xs_ref[...]
    lhs = xs_ref[:, pl.ds(j * n_rows, n_rows)]
    a = lax.dot_general(lhs, xs, (((0,), (0,)), ((), ())),
                        preferred_element_type=jnp.float32)
    a = a / math.sqrt(c)
    r = jnp.maximum(a, 0.0)
    e1 = jnp.exp(r - jnp.max(r, axis=-1, keepdims=True))
    s = e1 / jnp.sum(e1, axis=-1, keepdims=True)
    e2 = jnp.exp(s - jnp.max(s, axis=-1, keepdims=True))
    w = e2 / jnp.sum(e2, axis=-1, keepdims=True)

    bits = lax.bitcast_convert_type(w, jnp.int32)

    w_min = jnp.min(w, axis=-1, keepdims=True)
    t0 = lax.bitcast_convert_type(w_min, jnp.int32)
    gp = jnp.sum((bits > t0).astype(jnp.int32), axis=-1, keepdims=True)

    def full_search():
        def search(i, carry):
            lo, hi = carry
            mid = (lo + hi) >> 1
            cnt = jnp.sum((bits >= mid).astype(jnp.int32), axis=-1,
                          keepdims=True)
            ge = cnt >= k
            return jnp.where(ge, mid, lo), jnp.where(ge, hi, mid)

        lo0 = jnp.zeros((n_rows, 1), jnp.int32)
        hi0 = jnp.full((n_rows, 1), 0x3F800001, jnp.int32)
        t = lax.fori_loop(0, n_iters, search, (lo0, hi0))[0]
        g = jnp.sum((bits > t).astype(jnp.int32), axis=-1, keepdims=True)
        return t, g

    t, g = lax.cond(jnp.any(gp >= k), full_search, lambda: (t0, gp))

    gt = bits > t
    eq = bits == t
    z = eq.astype(jnp.int32)
    cum = z
    sh = 1
    while sh < n:
        cum = cum + lax.concatenate(
            [jnp.zeros((n_rows, sh), jnp.int32), cum[:, : n - sh]], 1)
        sh *= 2
    pc = cum - z
    keep = gt | (eq & (pc < (k - g)))
    out_ref[0] = jnp.where(keep, w, 0.0)


def kernel(x):
    b, c, n, t = x.shape
    k = int(n * 0.8)
    n_rows = 512 if n % 512 == 0 else n
    xt = jnp.transpose(x, (0, 3, 1, 2))
    body = functools.partial(_body, n_rows=n_rows, n=n, c=c, k=k, n_iters=30)
    return pl.pallas_call(
        body,
        grid=(b, n // n_rows),
        in_specs=[pl.BlockSpec((1, t, c, n), lambda bi, ji: (bi, 0, 0, 0))],
        out_specs=pl.BlockSpec((1, n_rows, n), lambda bi, ji: (bi, ji, 0)),
        out_shape=jax.ShapeDtypeStruct((b, n, n), jnp.float32),
        scratch_shapes=[pltpu.VMEM((c, n), jnp.float32)],
    )(xt)


# speedup vs baseline: 3.0890x; 1.3762x over previous
import functools
import math

import jax
import jax.numpy as jnp
from jax import lax
from jax.experimental import pallas as pl
from jax.experimental.pallas import tpu as pltpu


def _body(x_ref, sut_ref, out_ref, xs_ref, *, n_rows, n, c, k, n_iters):
    j = pl.program_id(1)

    @pl.when(j == 0)
    def _():
        xs_ref[...] = jnp.sum(x_ref[0], axis=0)

    xs = xs_ref[...]
    lhs = xs_ref[:, pl.ds(j * n_rows, n_rows)]
    a = lax.dot_general(lhs, xs, (((0,), (0,)), ((), ())),
                        preferred_element_type=jnp.float32)
    a = a / math.sqrt(c)
    r = jnp.maximum(a, 0.0)
    e1 = jnp.exp(r - jnp.max(r, axis=-1, keepdims=True))
    s = e1 / jnp.sum(e1, axis=-1, keepdims=True)
    e2 = jnp.exp(s - jnp.max(s, axis=-1, keepdims=True))
    w = e2 / jnp.sum(e2, axis=-1, keepdims=True)

    bits = lax.bitcast_convert_type(w, jnp.int32)

    w_min = jnp.min(w, axis=-1, keepdims=True)
    t0 = lax.bitcast_convert_type(w_min, jnp.int32)
    gp = jnp.sum((bits > t0).astype(jnp.int32), axis=-1, keepdims=True)

    def full_search():
        def search(i, carry):
            lo, hi = carry
            mid = (lo + hi) >> 1
            cnt = jnp.sum((bits >= mid).astype(jnp.int32), axis=-1,
                          keepdims=True)
            ge = cnt >= k
            return jnp.where(ge, mid, lo), jnp.where(ge, hi, mid)

        lo0 = jnp.zeros((n_rows, 1), jnp.int32)
        hi0 = jnp.full((n_rows, 1), 0x3F800001, jnp.int32)
        t = lax.fori_loop(0, n_iters, search, (lo0, hi0))[0]
        g = jnp.sum((bits > t).astype(jnp.int32), axis=-1, keepdims=True)
        return t, g

    t, g = lax.cond(jnp.any(gp >= k), full_search, lambda: (t0, gp))

    gt = bits > t
    eq = bits == t
    z = eq.astype(jnp.bfloat16)
    pc = lax.dot_general(z, sut_ref[...], (((1,), (0,)), ((), ())),
                         preferred_element_type=jnp.float32)
    keep = gt | (eq & (pc < (k - g).astype(jnp.float32)))
    out_ref[0] = jnp.where(keep, w, 0.0)


def kernel(x):
    b, c, n, t = x.shape
    k = int(n * 0.8)
    n_rows = 512 if n % 512 == 0 else n
    xt = jnp.transpose(x, (0, 3, 1, 2))
    sut = (jnp.arange(n)[:, None] < jnp.arange(n)[None, :]).astype(jnp.bfloat16)
    body = functools.partial(_body, n_rows=n_rows, n=n, c=c, k=k, n_iters=30)
    return pl.pallas_call(
        body,
        grid=(b, n // n_rows),
        in_specs=[pl.BlockSpec((1, t, c, n), lambda bi, ji: (bi, 0, 0, 0)),
                  pl.BlockSpec((n, n), lambda bi, ji: (0, 0))],
        out_specs=pl.BlockSpec((1, n_rows, n), lambda bi, ji: (bi, ji, 0)),
        out_shape=jax.ShapeDtypeStruct((b, n, n), jnp.float32),
        scratch_shapes=[pltpu.VMEM((c, n), jnp.float32)],
    )(xt, sut)
